# baseline (speedup 1.0000x reference)
import functools

import jax
import jax.numpy as jnp
from jax import lax
from jax.experimental import pallas as pl
from jax.experimental.pallas import tpu as pltpu
from jax.experimental.pallas import tpu_sc as plsc

ROWS = 4096
COLS = 32768
K = 200

NUM_CORES = 2
NUM_SUBCORES = 16
NUM_WORKERS = NUM_CORES * NUM_SUBCORES

ROWS_PER_W = ROWS // NUM_WORKERS
ELEMS_PER_W = ROWS_PER_W * K
GATHER_W = 128
N_GATHERS = ELEMS_PER_W // GATHER_W
FLIGHT = 8


def _body(tens_hbm, idx_hbm, out_hbm, raw_v, idx_v, out_v, sem):
    wid = lax.axis_index("s") * NUM_CORES + lax.axis_index("c")
    base = wid * ELEMS_PER_W
    row0 = wid * ROWS_PER_W

    pltpu.sync_copy(idx_hbm.at[pl.ds(base, ELEMS_PER_W)], raw_v)

    chunk_starts = tuple(min(c * 16, K - 16) for c in range((K + 15) // 16))

    def row_body(r, carry):
        off = (row0 + r) * COLS
        rb = r * K
        for s in chunk_starts:
            sl = pl.ds(rb + s, 16)
            idx_v[sl] = raw_v[sl] + off
        return carry

    lax.fori_loop(0, ROWS_PER_W, row_body, 0)

    def gather_body(g, carry):
        copies = []
        for f in range(FLIGHT):
            j = g * FLIGHT + f
            sl = pl.ds(j * GATHER_W, GATHER_W)
            copies.append(
                pltpu.async_copy(tens_hbm.at[idx_v.at[sl]], out_v.at[sl], sem)
            )
        for cp in copies:
            cp.wait()
        return carry

    lax.fori_loop(0, N_GATHERS // FLIGHT, gather_body, 0)

    pltpu.sync_copy(out_v, out_hbm.at[pl.ds(base, ELEMS_PER_W)])


@jax.jit
def _gather_flat(tensor_flat, idx_flat):
    mesh = plsc.VectorSubcoreMesh(core_axis_name="c", subcore_axis_name="s")
    fn = functools.partial(
        pl.kernel,
        mesh=mesh,
        out_type=jax.ShapeDtypeStruct((ROWS * K,), jnp.float32),
        scratch_types=[
            pltpu.VMEM((ELEMS_PER_W,), jnp.int32),
            pltpu.VMEM((ELEMS_PER_W,), jnp.int32),
            pltpu.VMEM((ELEMS_PER_W,), jnp.float32),
            pltpu.SemaphoreType.DMA,
        ],
    )(_body)
    return fn(tensor_flat, idx_flat)


def kernel(tensor, indices):
    tensor_flat = tensor.reshape(-1)
    idx_flat = indices.astype(jnp.int32).reshape(-1)
    out = _gather_flat(tensor_flat, idx_flat)
    return out.reshape(ROWS, K)


# speedup vs baseline: 1.0372x; 1.0372x over previous
import functools

import jax
import jax.numpy as jnp
from jax import lax
from jax.experimental import pallas as pl
from jax.experimental.pallas import tpu as pltpu
from jax.experimental.pallas import tpu_sc as plsc

ROWS = 4096
COLS = 32768
K = 200

NUM_CORES = 2
NUM_SUBCORES = 16
NUM_WORKERS = NUM_CORES * NUM_SUBCORES

ELEMS_PER_W = ROWS * K // NUM_WORKERS
N_CHUNKS = ELEMS_PER_W // 16


def _body(tens_hbm, idx_hbm, out_hbm, raw_v, idx_v, out_v, sem):
    wid = lax.axis_index("s") * NUM_CORES + lax.axis_index("c")
    ebase = wid * ELEMS_PER_W

    pltpu.sync_copy(idx_hbm.at[pl.ds(ebase, ELEMS_PER_W)], raw_v)

    iota = lax.iota(jnp.int32, 16)
    inv_k = jnp.float32(1.0 / K)

    def chunk_body(t, carry):
        sl = pl.ds(t * 16, 16)
        g = (ebase + t * 16) + iota
        row = ((g.astype(jnp.float32) + 0.5) * inv_k).astype(jnp.int32)
        idx_v[sl] = raw_v[sl] + row * COLS
        return carry

    lax.fori_loop(0, N_CHUNKS, chunk_body, 0)

    pltpu.async_copy(tens_hbm.at[idx_v], out_v, sem).wait()

    pltpu.sync_copy(out_v, out_hbm.at[pl.ds(ebase, ELEMS_PER_W)])


@jax.jit
def _gather_flat(tensor_flat, idx_flat):
    mesh = plsc.VectorSubcoreMesh(core_axis_name="c", subcore_axis_name="s")
    fn = functools.partial(
        pl.kernel,
        mesh=mesh,
        out_type=jax.ShapeDtypeStruct((ROWS * K,), jnp.float32),
        scratch_types=[
            pltpu.VMEM((ELEMS_PER_W,), jnp.int32),
            pltpu.VMEM((ELEMS_PER_W,), jnp.int32),
            pltpu.VMEM((ELEMS_PER_W,), jnp.float32),
            pltpu.SemaphoreType.DMA,
        ],
    )(_body)
    return fn(tensor_flat, idx_flat)


def kernel(tensor, indices):
    tensor_flat = tensor.reshape(-1)
    idx_flat = indices.astype(jnp.int32).reshape(-1)
    out = _gather_flat(tensor_flat, idx_flat)
    return out.reshape(ROWS, K)


# speedup vs baseline: 6.2673x; 6.0428x over previous
import functools

import jax
import jax.numpy as jnp
from jax import lax
from jax.experimental import pallas as pl
from jax.experimental.pallas import tpu as pltpu
from jax.experimental.pallas import tpu_sc as plsc

ROWS = 4096
COLS = 32768
K = 200
KP = 208

NUM_CORES = 2
NUM_SUBCORES = 16
NUM_WORKERS = NUM_CORES * NUM_SUBCORES

ROWS_PER_W = ROWS // NUM_WORKERS
ELEMS_PER_W = ROWS_PER_W * KP
CHUNKS_PER_ROW = KP // 16

SUB = 8
LANES = 128
TILE_ELEMS = SUB * LANES
ROWBLK_ELEMS = TILE_ELEMS * (COLS // LANES)


def _body(tens_hbm, idx_hbm, out_hbm, idx_v, val_v, sem):
    wid = lax.axis_index("s") * NUM_CORES + lax.axis_index("c")
    ebase = wid * ELEMS_PER_W
    rbase = wid * ROWS_PER_W

    pltpu.sync_copy(idx_hbm.at[pl.ds(ebase, ELEMS_PER_W)], idx_v)

    def per_row(t, carry):
        i = rbase + t
        rowpart = (i >> 3) * ROWBLK_ELEMS + (i & 7) * LANES
        for c in range(CHUNKS_PER_ROW):
            o = t * KP + c * 16
            j = idx_v[pl.ds(o, 16)]
            idx_v[pl.ds(o, 16)] = j + (j >> 7) * (TILE_ELEMS - LANES) + rowpart
        return carry

    lax.fori_loop(0, ROWS_PER_W, per_row, 0)

    pltpu.async_copy(tens_hbm.at[idx_v], val_v, sem).wait()
    pltpu.sync_copy(val_v, out_hbm.at[pl.ds(ebase, ELEMS_PER_W)])


@jax.jit
def _gather2d(tens_flat, idx_flat):
    mesh = plsc.VectorSubcoreMesh(core_axis_name="c", subcore_axis_name="s")
    fn = functools.partial(
        pl.kernel,
        mesh=mesh,
        out_type=jax.ShapeDtypeStruct((ROWS * KP,), jnp.float32),
        scratch_types=[
            pltpu.VMEM((ELEMS_PER_W,), jnp.int32),
            pltpu.VMEM((ELEMS_PER_W,), jnp.float32),
            pltpu.SemaphoreType.DMA,
        ],
    )(_body)
    return fn(tens_flat, idx_flat)


def kernel(tensor, indices):
    idx = jnp.pad(indices.astype(jnp.int32), ((0, 0), (0, KP - K)))
    flat = tensor.reshape(ROWS // SUB, SUB, COLS // LANES, LANES)
    flat = flat.transpose(0, 2, 1, 3).reshape(-1)
    out = _gather2d(flat, idx.reshape(-1))
    return out.reshape(ROWS, KP)[:, :K]


# speedup vs baseline: 6.4753x; 1.0332x over previous
import functools

import jax
import jax.numpy as jnp
from jax import lax
from jax.experimental import pallas as pl
from jax.experimental.pallas import tpu as pltpu
from jax.experimental.pallas import tpu_sc as plsc

ROWS = 4096
COLS = 32768
K = 200
KP = 208

NUM_CORES = 2
NUM_SUBCORES = 16
NUM_WORKERS = NUM_CORES * NUM_SUBCORES

ROWS_PER_W = ROWS // NUM_WORKERS
ELEMS_PER_W = ROWS_PER_W * KP
CHUNKS_PER_ROW = KP // 16

SUB = 8
LANES = 128
TILE_ELEMS = SUB * LANES
ROWBLK_ELEMS = TILE_ELEMS * (COLS // LANES)


NUM_GROUPS = 4
ROWS_PER_G = ROWS_PER_W // NUM_GROUPS
ELEMS_PER_G = ROWS_PER_G * KP


def _body(tens_hbm, idx_hbm, out_hbm, idx_v, val_v, sem_wb, *sems):
    wid = lax.axis_index("s") * NUM_CORES + lax.axis_index("c")
    ebase = wid * ELEMS_PER_W
    rbase = wid * ROWS_PER_W

    pltpu.sync_copy(idx_hbm.at[pl.ds(ebase, ELEMS_PER_W)], idx_v)

    streams = []
    for g in range(NUM_GROUPS):
        def per_row(t, carry):
            i = rbase + t
            rowpart = (i >> 3) * ROWBLK_ELEMS + (i & 7) * LANES
            for c in range(CHUNKS_PER_ROW):
                o = t * KP + c * 16
                j = idx_v[pl.ds(o, 16)]
                idx_v[pl.ds(o, 16)] = j + (j >> 7) * (TILE_ELEMS - LANES) + rowpart
            return carry

        lax.fori_loop(g * ROWS_PER_G, (g + 1) * ROWS_PER_G, per_row, 0)
        gs = pl.ds(g * ELEMS_PER_G, ELEMS_PER_G)
        streams.append(pltpu.async_copy(
            tens_hbm.at[idx_v.at[gs]], val_v.at[gs], sems[g]))

    wbs = []
    for g in range(NUM_GROUPS):
        streams[g].wait()
        gs = pl.ds(g * ELEMS_PER_G, ELEMS_PER_G)
        wbs.append(pltpu.async_copy(
            val_v.at[gs], out_hbm.at[pl.ds(ebase + g * ELEMS_PER_G, ELEMS_PER_G)],
            sem_wb))
    for wb in wbs:
        wb.wait()


@jax.jit
def _gather2d(tens_flat, idx_flat):
    mesh = plsc.VectorSubcoreMesh(core_axis_name="c", subcore_axis_name="s")
    fn = functools.partial(
        pl.kernel,
        mesh=mesh,
        out_type=jax.ShapeDtypeStruct((ROWS * KP,), jnp.float32),
        scratch_types=[
            pltpu.VMEM((ELEMS_PER_W,), jnp.int32),
            pltpu.VMEM((ELEMS_PER_W,), jnp.float32),
        ] + [pltpu.SemaphoreType.DMA] * (NUM_GROUPS + 1),
    )(_body)
    return fn(tens_flat, idx_flat)


def kernel(tensor, indices):
    idx = jnp.pad(indices.astype(jnp.int32), ((0, 0), (0, KP - K)))
    flat = tensor.reshape(ROWS // SUB, SUB, COLS // LANES, LANES)
    flat = flat.transpose(0, 2, 1, 3).reshape(-1)
    out = _gather2d(flat, idx.reshape(-1))
    return out.reshape(ROWS, KP)[:, :K]


# speedup vs baseline: 6.5005x; 1.0039x over previous
import functools

import jax
import jax.numpy as jnp
from jax import lax
from jax.experimental import pallas as pl
from jax.experimental.pallas import tpu as pltpu
from jax.experimental.pallas import tpu_sc as plsc

ROWS = 4096
COLS = 32768
K = 200
KP = 208

NUM_CORES = 2
NUM_SUBCORES = 16
NUM_WORKERS = NUM_CORES * NUM_SUBCORES

ROWS_PER_W = ROWS // NUM_WORKERS
ELEMS_PER_W = ROWS_PER_W * KP
CHUNKS_PER_ROW = KP // 16

SUB = 8
LANES = 128
TILE_ELEMS = SUB * LANES
ROWBLK_ELEMS = TILE_ELEMS * (COLS // LANES)

NUM_GROUPS = 8
ROWS_PER_G = ROWS_PER_W // NUM_GROUPS
ELEMS_PER_G = ROWS_PER_G * KP


def _body(tens_hbm, idx_hbm, out_hbm, idx_v, val_v, sem_wb, *sems):
    wid = lax.axis_index("s") * NUM_CORES + lax.axis_index("c")
    ebase = wid * ELEMS_PER_W
    rbase = wid * ROWS_PER_W

    stages = []
    for g in range(NUM_GROUPS):
        gs = pl.ds(g * ELEMS_PER_G, ELEMS_PER_G)
        stages.append(pltpu.async_copy(
            idx_hbm.at[pl.ds(ebase + g * ELEMS_PER_G, ELEMS_PER_G)],
            idx_v.at[gs], sems[g]))

    streams = []
    for g in range(NUM_GROUPS):
        stages[g].wait()

        def per_row(t, carry):
            i = rbase + t
            rowpart = (i >> 3) * ROWBLK_ELEMS + (i & 7) * LANES
            for c in range(CHUNKS_PER_ROW):
                o = t * KP + c * 16
                j = idx_v[pl.ds(o, 16)]
                idx_v[pl.ds(o, 16)] = j + (j >> 7) * (TILE_ELEMS - LANES) + rowpart
            return carry

        lax.fori_loop(g * ROWS_PER_G, (g + 1) * ROWS_PER_G, per_row, 0)
        gs = pl.ds(g * ELEMS_PER_G, ELEMS_PER_G)
        streams.append(pltpu.async_copy(
            tens_hbm.at[idx_v.at[gs]], val_v.at[gs], sems[g]))

    wbs = []
    for g in range(NUM_GROUPS):
        streams[g].wait()
        gs = pl.ds(g * ELEMS_PER_G, ELEMS_PER_G)
        wbs.append(pltpu.async_copy(
            val_v.at[gs], out_hbm.at[pl.ds(ebase + g * ELEMS_PER_G, ELEMS_PER_G)],
            sem_wb))
    for wb in wbs:
        wb.wait()


@jax.jit
def _gather2d(tens_flat, idx_flat):
    mesh = plsc.VectorSubcoreMesh(core_axis_name="c", subcore_axis_name="s")
    fn = functools.partial(
        pl.kernel,
        mesh=mesh,
        out_type=jax.ShapeDtypeStruct((ROWS * KP,), jnp.float32),
        scratch_types=[
            pltpu.VMEM((ELEMS_PER_W,), jnp.int32),
            pltpu.VMEM((ELEMS_PER_W,), jnp.float32),
        ] + [pltpu.SemaphoreType.DMA] * (NUM_GROUPS + 1),
    )(_body)
    return fn(tens_flat, idx_flat)


def kernel(tensor, indices):
    idx = jnp.pad(indices.astype(jnp.int32), ((0, 0), (0, KP - K)))
    flat = tensor.reshape(ROWS // SUB, SUB, COLS // LANES, LANES)
    flat = flat.transpose(0, 2, 1, 3).reshape(-1)
    out = _gather2d(flat, idx.reshape(-1))
    return out.reshape(ROWS, KP)[:, :K]
